# baseline (speedup 1.0000x reference)
import functools

import jax
import jax.numpy as jnp
from jax import lax
from jax.experimental import pallas as pl
from jax.experimental.pallas import tpu as pltpu
from jax.experimental.pallas import tpu_sc as plsc


def kernel(input_seqs, pos_emb):
    B, L = input_seqs.shape
    Lk, E = pos_emb.shape
    D = Lk * E
    flat = pos_emb.reshape(D)

    info = plsc.get_sparse_core_info()
    NC, NS = info.num_cores, info.num_subcores
    NW = NC * NS
    bpw = B // NW
    SP = 256
    per_tec = SP // NS
    n_out = bpw // SP

    mesh = plsc.VectorSubcoreMesh(core_axis_name="c", subcore_axis_name="s")

    @functools.partial(
        pl.kernel,
        mesh=mesh,
        out_type=jax.ShapeDtypeStruct((B, D), jnp.float32),
        scratch_types=[
            pltpu.VMEM_SHARED((SP, D), jnp.float32),
            pltpu.SemaphoreType.DMA,
            pltpu.SemaphoreType.DMA,
        ],
    )
    def k(emb_hbm, out_hbm, shared, fill_sem, out_sem):
        sid = lax.axis_index("s")
        wid = sid * NC + lax.axis_index("c")
        base = wid * bpw

        fills = [
            pltpu.async_copy(emb_hbm, shared.at[sid * per_tec + i], fill_sem)
            for i in range(per_tec)
        ]
        for c in fills:
            c.wait()
        plsc.subcore_barrier()

        outs = [
            pltpu.async_copy(shared, out_hbm.at[pl.ds(base + j * SP, SP)], out_sem)
            for j in range(n_out)
        ]
        for c in outs:
            c.wait()

    out = k(flat)
    return out.reshape(B, L, E)


# speedup vs baseline: 1.0287x; 1.0287x over previous
import functools

import jax
import jax.numpy as jnp
from jax import lax
from jax.experimental import pallas as pl
from jax.experimental.pallas import tpu as pltpu
from jax.experimental.pallas import tpu_sc as plsc


def kernel(input_seqs, pos_emb):
    B, L = input_seqs.shape
    Lk, E = pos_emb.shape
    D = Lk * E
    flat = pos_emb.reshape(D)

    info = plsc.get_sparse_core_info()
    NC = info.num_cores
    rows_per_sc = B // NC
    SP = 128
    n_out = rows_per_sc // SP
    WINDOW = 8

    mesh = plsc.ScalarSubcoreMesh(axis_name="c", num_cores=NC)

    @functools.partial(
        pl.kernel,
        mesh=mesh,
        out_type=jax.ShapeDtypeStruct((B, D), jnp.float32),
        scratch_types=[
            pltpu.VMEM_SHARED((SP, D), jnp.float32),
            pltpu.SemaphoreType.DMA,
            pltpu.SemaphoreType.DMA,
        ],
    )
    def k(emb_hbm, out_hbm, shared, fill_sem, out_sem):
        cid = lax.axis_index("c")
        base = cid * rows_per_sc

        fills = []
        for i in range(SP):
            if len(fills) == 16:
                fills.pop(0).wait()
            fills.append(pltpu.async_copy(emb_hbm, shared.at[i], fill_sem))
        for c in fills:
            c.wait()

        pending = []
        for j in range(n_out):
            if len(pending) == WINDOW:
                pending.pop(0).wait()
            pending.append(
                pltpu.async_copy(
                    shared, out_hbm.at[pl.ds(base + j * SP, SP)], out_sem
                )
            )
        for c in pending:
            c.wait()

    out = k(flat)
    return out.reshape(B, L, E)


# speedup vs baseline: 1.1396x; 1.1079x over previous
import functools

import jax
import jax.numpy as jnp
from jax import lax
from jax.experimental import pallas as pl
from jax.experimental.pallas import tpu as pltpu
from jax.experimental.pallas import tpu_sc as plsc


def kernel(input_seqs, pos_emb):
    B, L = input_seqs.shape
    Lk, E = pos_emb.shape
    D = Lk * E
    flat = pos_emb.reshape(D)

    info = plsc.get_sparse_core_info()
    NC, NS = info.num_cores, info.num_subcores
    NW = NC * NS
    bpw = B // NW
    K = 16
    n_dma = bpw // K
    WINDOW = 8

    mesh = plsc.VectorSubcoreMesh(core_axis_name="c", subcore_axis_name="s")

    @functools.partial(
        pl.kernel,
        mesh=mesh,
        out_type=jax.ShapeDtypeStruct((B, D), jnp.float32),
        scratch_types=[
            pltpu.VMEM((K, D), jnp.float32),
            pltpu.SemaphoreType.DMA,
            pltpu.SemaphoreType.DMA,
        ],
    )
    def k(emb_hbm, out_hbm, buf, fill_sem, out_sem):
        wid = lax.axis_index("s") * NC + lax.axis_index("c")
        base = wid * bpw

        fills = [pltpu.async_copy(emb_hbm, buf.at[i], fill_sem) for i in range(K)]
        for c in fills:
            c.wait()

        pending = []
        for i in range(n_dma):
            if len(pending) == WINDOW:
                pending.pop(0).wait()
            pending.append(
                pltpu.async_copy(buf, out_hbm.at[pl.ds(base + i * K, K)], out_sem)
            )
        for c in pending:
            c.wait()

    out = k(flat)
    return out.reshape(B, L, E)
